# baseline (speedup 1.0000x reference)
import functools

import jax
import jax.numpy as jnp
from jax import lax
from jax.experimental import pallas as pl
from jax.experimental.pallas import tpu as pltpu
from jax.experimental.pallas import tpu_sc as plsc

_POOL = 16384
_B = 32
_S = 197
_C = 768
_K = 5
_L = 5
_BLK = 2048
_NB = _POOL // _BLK


def _sim_topk_body(x_ref, key_ref, sim_ref, idx_ref, rs_ref, xn_ref, rv_ref, ri_ref):
    i = pl.program_id(0)

    @pl.when(i == 0)
    def _():
        xm = jnp.sum(x_ref[...], axis=0) * (1.0 / _S)
        ssq = jnp.sum(xm * xm, axis=1, keepdims=True)
        xn_ref[...] = (xm * lax.rsqrt(jnp.maximum(ssq, 1e-12))).astype(jnp.bfloat16)
        rv_ref[...] = jnp.full((8, _B), -jnp.inf, jnp.float32)
        ri_ref[...] = jnp.full((8, _B), _POOL, jnp.int32)

    kb = key_ref[...]
    ssq = jnp.sum(kb * kb, axis=1, keepdims=True)
    knb = (kb * lax.rsqrt(jnp.maximum(ssq, 1e-12))).astype(jnp.bfloat16)
    xnb = xn_ref[...]
    simt_b = lax.dot_general(knb, xnb, (((1,), (1,)), ((), ())),
                             preferred_element_type=jnp.float32)
    sim_b = lax.dot_general(xnb, knb, (((1,), (1,)), ((), ())),
                            preferred_element_type=jnp.float32)
    sim_ref[:, pl.ds(i * _BLK, _BLK)] = sim_b

    _CH = 256
    rows0 = lax.broadcasted_iota(jnp.int32, (_CH, _B), 0) + i * _BLK
    cand_v = [rv_ref[pl.ds(0, _K)]]
    cand_i = [ri_ref[pl.ds(0, _K)]]
    for c in range(_BLK // _CH):
        bv = simt_b[c * _CH:(c + 1) * _CH, :]
        rws = rows0 + c * _CH
        for _ in range(_K):
            m = jnp.max(bv, axis=0, keepdims=True)
            am = jnp.min(jnp.where(bv == m, rws, _POOL), axis=0, keepdims=True)
            cand_v.append(m)
            cand_i.append(am)
            bv = jnp.where(rws == am, -jnp.inf, bv)

    allv = jnp.concatenate(cand_v, axis=0)
    alli = jnp.concatenate(cand_i, axis=0)
    new_v, new_i = [], []
    for _ in range(_K):
        m = jnp.max(allv, axis=0, keepdims=True)
        am = jnp.min(jnp.where(allv == m, alli, _POOL), axis=0, keepdims=True)
        new_v.append(m)
        new_i.append(am)
        allv = jnp.where(alli == am, -jnp.inf, allv)
    padf = jnp.full((8 - _K, _B), -jnp.inf, jnp.float32)
    padi = jnp.full((8 - _K, _B), _POOL, jnp.int32)
    rv_ref[...] = jnp.concatenate(new_v + [padf], axis=0)
    ri_ref[...] = jnp.concatenate(new_i + [padi], axis=0)

    @pl.when(i == _NB - 1)
    def _():
        padz = jnp.zeros((8 - _K, _B), jnp.int32)
        idx_ref[...] = jnp.concatenate(new_i + [padz], axis=0)
        rs_ref[0, 0] = jnp.sum(jnp.concatenate(new_v, axis=0)) * (1.0 / _B)


def _sim_topk(x_seq_major, prompt_key):
    return pl.pallas_call(
        _sim_topk_body,
        grid=(_NB,),
        in_specs=[
            pl.BlockSpec((_S, _B, _C), lambda i: (0, 0, 0)),
            pl.BlockSpec((_BLK, _C), lambda i: (i, 0)),
        ],
        out_specs=[
            pl.BlockSpec((_B, _POOL), lambda i: (0, 0)),
            pl.BlockSpec((8, _B), lambda i: (0, 0)),
            pl.BlockSpec(memory_space=pltpu.MemorySpace.SMEM),
        ],
        out_shape=[
            jax.ShapeDtypeStruct((_B, _POOL), jnp.float32),
            jax.ShapeDtypeStruct((8, _B), jnp.int32),
            jax.ShapeDtypeStruct((1, 1), jnp.float32),
        ],
        scratch_shapes=[
            pltpu.VMEM((_B, _C), jnp.bfloat16),
            pltpu.VMEM((8, _B), jnp.float32),
            pltpu.VMEM((8, _B), jnp.int32),
        ],
    )(x_seq_major, prompt_key)


def _sc_gather(table_flat, idxt):
    info = plsc.get_sparse_core_info()
    mesh = plsc.VectorSubcoreMesh(core_axis_name="c", subcore_axis_name="s")

    @functools.partial(
        pl.kernel,
        mesh=mesh,
        out_type=jax.ShapeDtypeStruct((_K * _L, _B, _C), jnp.float32),
        scratch_types=[
            pltpu.VMEM((_B,), jnp.int32),
            pltpu.VMEM((_B, _C), jnp.float32),
            pltpu.SemaphoreType.DMA,
        ],
    )
    def gather_k(table_hbm, idx_hbm, out_hbm, myidx_v, rows_v, sem):
        t = lax.axis_index("s") * info.num_cores + lax.axis_index("c")

        @pl.when(t < _K * _L)
        def _():
            k = t // _L
            l = t % _L
            pltpu.sync_copy(idx_hbm.at[k], myidx_v)
            off = l * _POOL
            myidx_v[pl.ds(0, 16)] = myidx_v[pl.ds(0, 16)] + off
            myidx_v[pl.ds(16, 16)] = myidx_v[pl.ds(16, 16)] + off
            pltpu.async_copy(table_hbm.at[myidx_v], rows_v, sem).wait()
            pltpu.sync_copy(rows_v, out_hbm.at[t])

    return gather_k(table_flat, idxt)


def kernel(x_embed, prompt, prompt_key):
    x_seq_major = jnp.transpose(x_embed, (1, 0, 2))
    sim, idxt, rs = _sim_topk(x_seq_major, prompt_key)
    table_flat = jnp.transpose(prompt, (0, 2, 1, 3)).reshape(_L * _POOL, _C)
    gathered = _sc_gather(table_flat, idxt)
    batched_prompt = jnp.transpose(gathered, (1, 0, 2)).reshape(1, _B, _K * _L, _C)
    return batched_prompt, rs[0, 0], sim


# speedup vs baseline: 1.0182x; 1.0182x over previous
import functools

import jax
import jax.numpy as jnp
from jax import lax
from jax.experimental import pallas as pl
from jax.experimental.pallas import tpu as pltpu
from jax.experimental.pallas import tpu_sc as plsc

_POOL = 16384
_B = 32
_S = 197
_C = 768
_K = 5
_L = 5
_BLK = 2048
_NB = _POOL // _BLK


def _sim_topk_body(x_ref, key_ref, sim_ref, idx_ref, rs_ref, xn_ref, rv_ref, ri_ref):
    i = pl.program_id(0)

    @pl.when(i == 0)
    def _():
        xm = jnp.sum(x_ref[...], axis=0) * (1.0 / _S)
        ssq = jnp.sum(xm * xm, axis=1, keepdims=True)
        xn_ref[...] = (xm * lax.rsqrt(jnp.maximum(ssq, 1e-12))).astype(jnp.bfloat16)
        rv_ref[...] = jnp.full((8, _B), -jnp.inf, jnp.float32)
        ri_ref[...] = jnp.full((8, _B), _POOL, jnp.int32)

    kb = key_ref[...]
    ssq = jnp.sum(kb * kb, axis=1, keepdims=True)
    knb = (kb * lax.rsqrt(jnp.maximum(ssq, 1e-12))).astype(jnp.bfloat16)
    xnb = xn_ref[...]
    simt_b = lax.dot_general(knb, xnb, (((1,), (1,)), ((), ())),
                             preferred_element_type=jnp.float32)
    sim_b = lax.dot_general(xnb, knb, (((1,), (1,)), ((), ())),
                            preferred_element_type=jnp.float32)
    sim_ref[:, pl.ds(i * _BLK, _BLK)] = sim_b

    _CH = 512
    rows0 = lax.broadcasted_iota(jnp.int32, (_CH, _B), 0) + i * _BLK
    cand_v = [rv_ref[pl.ds(0, _K)]]
    cand_i = [ri_ref[pl.ds(0, _K)]]
    for c in range(_BLK // _CH):
        bv = simt_b[c * _CH:(c + 1) * _CH, :]
        rws = rows0 + c * _CH
        for _ in range(_K):
            m = jnp.max(bv, axis=0, keepdims=True)
            am = jnp.min(jnp.where(bv == m, rws, _POOL), axis=0, keepdims=True)
            cand_v.append(m)
            cand_i.append(am)
            bv = jnp.where(rws == am, -jnp.inf, bv)

    allv = jnp.concatenate(cand_v, axis=0)
    alli = jnp.concatenate(cand_i, axis=0)
    new_v, new_i = [], []
    for _ in range(_K):
        m = jnp.max(allv, axis=0, keepdims=True)
        am = jnp.min(jnp.where(allv == m, alli, _POOL), axis=0, keepdims=True)
        new_v.append(m)
        new_i.append(am)
        allv = jnp.where(alli == am, -jnp.inf, allv)
    padf = jnp.full((8 - _K, _B), -jnp.inf, jnp.float32)
    padi = jnp.full((8 - _K, _B), _POOL, jnp.int32)
    rv_ref[...] = jnp.concatenate(new_v + [padf], axis=0)
    ri_ref[...] = jnp.concatenate(new_i + [padi], axis=0)

    @pl.when(i == _NB - 1)
    def _():
        padz = jnp.zeros((8 - _K, _B), jnp.int32)
        idx_ref[...] = jnp.concatenate(new_i + [padz], axis=0)
        rs_ref[0, 0] = jnp.sum(jnp.concatenate(new_v, axis=0)) * (1.0 / _B)


def _sim_topk(x_seq_major, prompt_key):
    return pl.pallas_call(
        _sim_topk_body,
        grid=(_NB,),
        in_specs=[
            pl.BlockSpec((_S, _B, _C), lambda i: (0, 0, 0)),
            pl.BlockSpec((_BLK, _C), lambda i: (i, 0)),
        ],
        out_specs=[
            pl.BlockSpec((_B, _POOL), lambda i: (0, 0)),
            pl.BlockSpec((8, _B), lambda i: (0, 0)),
            pl.BlockSpec(memory_space=pltpu.MemorySpace.SMEM),
        ],
        out_shape=[
            jax.ShapeDtypeStruct((_B, _POOL), jnp.float32),
            jax.ShapeDtypeStruct((8, _B), jnp.int32),
            jax.ShapeDtypeStruct((1, 1), jnp.float32),
        ],
        scratch_shapes=[
            pltpu.VMEM((_B, _C), jnp.bfloat16),
            pltpu.VMEM((8, _B), jnp.float32),
            pltpu.VMEM((8, _B), jnp.int32),
        ],
    )(x_seq_major, prompt_key)


def _sc_gather(table_flat, idxt):
    info = plsc.get_sparse_core_info()
    mesh = plsc.VectorSubcoreMesh(core_axis_name="c", subcore_axis_name="s")

    @functools.partial(
        pl.kernel,
        mesh=mesh,
        out_type=jax.ShapeDtypeStruct((_K * _L, _B, _C), jnp.float32),
        scratch_types=[
            pltpu.VMEM((_B,), jnp.int32),
            pltpu.VMEM((_B, _C), jnp.float32),
            pltpu.SemaphoreType.DMA,
        ],
    )
    def gather_k(table_hbm, idx_hbm, out_hbm, myidx_v, rows_v, sem):
        t = lax.axis_index("s") * info.num_cores + lax.axis_index("c")

        @pl.when(t < _K * _L)
        def _():
            k = t // _L
            l = t % _L
            pltpu.sync_copy(idx_hbm.at[k], myidx_v)
            off = l * _POOL
            myidx_v[pl.ds(0, 16)] = myidx_v[pl.ds(0, 16)] + off
            myidx_v[pl.ds(16, 16)] = myidx_v[pl.ds(16, 16)] + off
            pltpu.async_copy(table_hbm.at[myidx_v], rows_v, sem).wait()
            pltpu.sync_copy(rows_v, out_hbm.at[t])

    return gather_k(table_flat, idxt)


def kernel(x_embed, prompt, prompt_key):
    x_seq_major = jnp.transpose(x_embed, (1, 0, 2))
    sim, idxt, rs = _sim_topk(x_seq_major, prompt_key)
    table_flat = jnp.transpose(prompt, (0, 2, 1, 3)).reshape(_L * _POOL, _C)
    gathered = _sc_gather(table_flat, idxt)
    batched_prompt = jnp.transpose(gathered, (1, 0, 2)).reshape(1, _B, _K * _L, _C)
    return batched_prompt, rs[0, 0], sim
